# baseline (speedup 1.0000x reference)
import functools

import jax
import jax.numpy as jnp
from jax import lax
from jax.experimental import pallas as pl
from jax.experimental.pallas import tpu as pltpu
from jax.experimental.pallas import tpu_sc as plsc

_NC = 2
_NS = 16
_NW = _NC * _NS
_CH = 250


def _seg_sum(table, src3, dst3, *, want_deg):
    NP, D = table.shape
    assert D % 16 == 0 and NP % (128 * _NS) == 0
    _, NB, CH = src3.shape
    RS = NP // _NS
    OP = ((CH + 15) // 16) * 16

    out_type = [jax.ShapeDtypeStruct((_NC, NP, D), jnp.float32)]
    scratch = [
        pltpu.VMEM_SHARED((NP, D), jnp.float32),
        pltpu.VMEM((NB, CH), jnp.int32),
        pltpu.VMEM((NB, CH), jnp.int32),
        pltpu.VMEM((128, D), jnp.float32),
        pltpu.VMEM((CH, D), jnp.float32),
        pltpu.VMEM((CH, D), jnp.float32),
        pltpu.SemaphoreType.DMA,
        pltpu.SemaphoreType.DMA,
    ]
    if want_deg:
        out_type.append(jax.ShapeDtypeStruct((_NC * NP,), jnp.float32))
        scratch.append(pltpu.VMEM_SHARED((NP,), jnp.float32))
        scratch.append(pltpu.VMEM((OP,), jnp.float32))
        scratch.append(pltpu.VMEM((RS,), jnp.float32))
        scratch.append(pltpu.SemaphoreType.DMA)

    mesh = plsc.VectorSubcoreMesh(core_axis_name="c", subcore_axis_name="s")

    def body(table_h, src_h, dst_h, out_h, *rest):
        if want_deg:
            (deg_h, acc_sh, src_v, dst_v, zbuf, b0, b1, semg, sems,
             deg_sh, ones_v, deg_v, semd) = rest
        else:
            acc_sh, src_v, dst_v, zbuf, b0, b1, semg, sems = rest
        c = lax.axis_index("c")
        s = lax.axis_index("s")
        wid = s * _NC + c
        r0 = s * RS

        zero16 = jnp.zeros((16,), jnp.float32)

        def zrow(i, carry):
            for k in range(D // 16):
                zbuf[i, pl.ds(k * 16, 16)] = zero16
            return carry

        lax.fori_loop(0, 128, zrow, 0)
        for q in range(RS // 128):
            pltpu.sync_copy(zbuf, acc_sh.at[pl.ds(r0 + q * 128, 128)])

        if want_deg:
            def zdeg(i, carry):
                deg_v[pl.ds(i * 16, 16)] = zero16
                return carry

            lax.fori_loop(0, RS // 16, zdeg, 0)
            pltpu.sync_copy(deg_v, deg_sh.at[pl.ds(r0, RS)])
            one16 = jnp.ones((16,), jnp.float32)
            for k in range(OP // 16):
                ones_v[pl.ds(k * 16, 16)] = one16

        pltpu.sync_copy(src_h.at[wid], src_v)
        pltpu.sync_copy(dst_h.at[wid], dst_v)

        plsc.subcore_barrier()

        bufs = [b0, b1]
        sd = [None] * NB
        gd = [None] * NB
        dd = [None] * NB
        gd[0] = pltpu.async_copy(table_h.at[src_v.at[0]], b0, semg)
        for j in range(NB):
            cur, nxt = bufs[j % 2], bufs[(j + 1) % 2]
            if j + 1 < NB:
                if j >= 1:
                    sd[j - 1].wait()
                gd[j + 1] = pltpu.async_copy(table_h.at[src_v.at[j + 1]],
                                             nxt, semg)
            gd[j].wait()
            sd[j] = pltpu.async_copy(cur, acc_sh.at[dst_v.at[j]], sems,
                                     add=True)
            if want_deg:
                dd[j] = pltpu.async_copy(ones_v.at[pl.ds(0, CH)],
                                         deg_sh.at[dst_v.at[j]], semd,
                                         add=True)
        sd[NB - 1].wait()
        if NB >= 2:
            sd[NB - 2].wait()
        if want_deg:
            for j in range(NB):
                dd[j].wait()

        plsc.subcore_barrier()

        pltpu.sync_copy(acc_sh.at[pl.ds(r0, RS)], out_h.at[c, pl.ds(r0, RS)])
        if want_deg:
            pltpu.sync_copy(deg_sh.at[pl.ds(r0, RS)], deg_v)
            pltpu.sync_copy(deg_v, deg_h.at[pl.ds(c * NP + r0, RS)])

    fn = pl.kernel(body, out_type=out_type, mesh=mesh, scratch_types=scratch,
                   compiler_params=pltpu.CompilerParams(
                       use_tc_tiling_on_sc=False))
    return fn(table, src3, dst3)


_BR = 1024


def _tc1(x, Wcat, bcat):
    NP, F = x.shape
    H2 = Wcat.shape[1]
    H = H2 // 2
    GN = NP // _BR

    def body(x_ref, w_ref, b_ref, y_ref, h1_ref):
        yh = jnp.dot(x_ref[...], w_ref[...],
                     preferred_element_type=jnp.float32) + b_ref[...]
        y_ref[...] = yh[:, :H]
        h1_ref[...] = yh[:, H:]

    return pl.pallas_call(
        body,
        grid=(GN,),
        in_specs=[pl.BlockSpec((_BR, F), lambda i: (i, 0)),
                  pl.BlockSpec((F, H2), lambda i: (0, 0)),
                  pl.BlockSpec((1, H2), lambda i: (0, 0))],
        out_specs=[pl.BlockSpec((_BR, H), lambda i: (i, 0)),
                   pl.BlockSpec((_BR, H), lambda i: (i, 0))],
        out_shape=[jax.ShapeDtypeStruct((NP, H), jnp.float32),
                   jax.ShapeDtypeStruct((NP, H), jnp.float32)],
    )(x, Wcat, bcat.reshape(1, H2))


def _tc2(y, h1, accA, degc, Tf2, tw, alpha):
    NP, H = y.shape
    T, M = tw.shape

    def body(y_ref, h1_ref, accA_ref, deg_ref, tf_ref, tw_ref, a_ref,
             yt_ref, yb_ref):
        a = a_ref[0, 0]
        deg = deg_ref[...]
        nbr = (accA_ref[0] + accA_ref[1]) / jnp.maximum(deg, 1.0)
        h = a * y_ref[...] + (1.0 - a) * nbr
        twv = tw_ref[...]
        twm = jnp.max(twv, axis=1, keepdims=True)
        we = jnp.exp(twv - twm)
        w = we / jnp.sum(we, axis=1, keepdims=True)
        tf = tf_ref[...]
        w_tile = jnp.tile(w, (1, T))
        col_t = lax.broadcasted_iota(jnp.int32, (T, T * M), 1) // M
        row_t = lax.broadcasted_iota(jnp.int32, (T, T * M), 0)
        B = jnp.where(col_t == row_t, w_tile, 0.0)
        Wt = jnp.dot(B, tf, preferred_element_type=jnp.float32)
        tn_row = jnp.sum(tf * tf, axis=1)[None, :]
        c1 = lax.dot_general(tn_row, B, (((1,), (1,)), ((), ())),
                             preferred_element_type=jnp.float32)
        hn = jnp.sum(h * h, axis=1, keepdims=True)
        cross = lax.dot_general(h, Wt, (((1,), (1,)), ((), ())),
                                preferred_element_type=jnp.float32)
        yt_ref[...] = hn + c1 - 2.0 * cross
        dinv = lax.rsqrt(deg + 1.0)
        yb_ref[...] = h1_ref[...] * dinv

    GN = NP // _BR
    return pl.pallas_call(
        body,
        grid=(GN,),
        in_specs=[pl.BlockSpec((_BR, H), lambda i: (i, 0)),
                  pl.BlockSpec((_BR, H), lambda i: (i, 0)),
                  pl.BlockSpec((2, _BR, H), lambda i: (0, i, 0)),
                  pl.BlockSpec((_BR, 1), lambda i: (i, 0)),
                  pl.BlockSpec(Tf2.shape, lambda i: (0, 0)),
                  pl.BlockSpec((T, M), lambda i: (0, 0)),
                  pl.BlockSpec((1, 1), lambda i: (0, 0))],
        out_specs=[pl.BlockSpec((_BR, T), lambda i: (i, 0)),
                   pl.BlockSpec((_BR, H), lambda i: (i, 0))],
        out_shape=[jax.ShapeDtypeStruct((NP, T), jnp.float32),
                   jax.ShapeDtypeStruct((NP, H), jnp.float32)],
    )(y, h1, accA, degc, Tf2, tw, alpha.reshape(1, 1))


def _tc3(accB, h1, degc, b1, yt, gamma, beta, W2, n_real):
    NP, H = h1.shape
    T = yt.shape[1]
    G = H + T
    C = W2.shape[1]
    CP = ((C + 15) // 16) * 16

    def body(accB_ref, h1_ref, deg_ref, b1_ref, yt_ref, g_ref, be_ref, w2_ref,
             xh_ref, h2_ref, h2b_ref):
        deg = deg_ref[...]
        dinv = lax.rsqrt(deg + 1.0)
        ssum = accB_ref[0] + accB_ref[1]
        z = jnp.maximum(dinv * ssum + h1_ref[...] * dinv * dinv + b1_ref[...],
                        0.0)
        xc = jnp.concatenate([z, yt_ref[...]], axis=1)
        xr = xc[:n_real]
        mu = jnp.mean(xr, axis=0, keepdims=True)
        d = xr - mu
        var = jnp.mean(d * d, axis=0, keepdims=True)
        xh = (xc - mu) * lax.rsqrt(var + 1e-5) * g_ref[...] + be_ref[...]
        xh_ref[...] = xh
        h2 = jnp.dot(xh, w2_ref[...], preferred_element_type=jnp.float32)
        h2_ref[...] = h2
        h2b = h2 * dinv
        h2b_ref[...] = jnp.concatenate(
            [h2b, jnp.zeros((h2b.shape[0], CP - C), jnp.float32)], axis=1)

    return pl.pallas_call(
        body,
        out_shape=[jax.ShapeDtypeStruct((NP, G), jnp.float32),
                   jax.ShapeDtypeStruct((NP, C), jnp.float32),
                   jax.ShapeDtypeStruct((NP, CP), jnp.float32)],
    )(accB, h1, degc, b1.reshape(1, H), yt, gamma.reshape(1, G),
      beta.reshape(1, G), W2)


def _tc4(accC, h2, degc, b2):
    NP, C = h2.shape
    CP = accC.shape[2]
    GN = NP // _BR

    def body(accC_ref, h2_ref, deg_ref, b2_ref, out_ref):
        deg = deg_ref[...]
        dinv = lax.rsqrt(deg + 1.0)
        acc = accC_ref[0, :, :C] + accC_ref[1, :, :C]
        out_ref[...] = dinv * acc + h2_ref[...] * dinv * dinv + b2_ref[...]

    return pl.pallas_call(
        body,
        grid=(GN,),
        in_specs=[pl.BlockSpec((2, _BR, CP), lambda i: (0, i, 0)),
                  pl.BlockSpec((_BR, C), lambda i: (i, 0)),
                  pl.BlockSpec((_BR, 1), lambda i: (i, 0)),
                  pl.BlockSpec((1, C), lambda i: (0, 0))],
        out_specs=pl.BlockSpec((_BR, C), lambda i: (i, 0)),
        out_shape=jax.ShapeDtypeStruct((NP, C), jnp.float32),
    )(accC, h2, degc, b2.reshape(1, C))


def kernel(x, edge_index, lin_W, lin_b, W1, b1, W2, b2, Tf, tw, alpha,
           gamma, beta):
    N, F = x.shape
    E = edge_index.shape[1]
    T, M, H = Tf.shape

    NP = ((N + 128 * _NS - 1) // (128 * _NS)) * (128 * _NS)
    EPT = E // _NW
    NCH = EPT // _CH
    src3 = edge_index[0].reshape(_NW, NCH, _CH)
    dst3 = edge_index[1].reshape(_NW, NCH, _CH)

    xp = jnp.pad(x, ((0, NP - N), (0, 0)))
    Wcat = jnp.concatenate([lin_W, W1], axis=1)
    bcat = jnp.concatenate([lin_b, jnp.zeros_like(b1)], axis=0)

    y, h1 = _tc1(xp, Wcat, bcat)
    accA, degp = _seg_sum(y, src3, dst3, want_deg=True)
    degc = (degp[:NP] + degp[NP:])[:, None]
    yt, yb = _tc2(y, h1, accA, degc, Tf.reshape(T * M, H), tw, alpha)
    (accB,) = _seg_sum(yb, src3, dst3, want_deg=False)
    xh, h2, h2b = _tc3(accB, h1, degc, b1, yt, gamma, beta, W2, N)
    (accCp,) = _seg_sum(h2b, src3, dst3, want_deg=False)
    out = _tc4(accCp, h2, degc, b2)
    return (out[:N], xh[:N])


# speedup vs baseline: 1.0203x; 1.0203x over previous
import functools

import jax
import jax.numpy as jnp
from jax import lax
from jax.experimental import pallas as pl
from jax.experimental.pallas import tpu as pltpu
from jax.experimental.pallas import tpu_sc as plsc

_NC = 2
_NS = 16
_NW = _NC * _NS
_CH = 250


def _seg_sum(table, src3, dst3, *, want_deg):
    NP, D = table.shape
    assert D % 16 == 0 and NP % (128 * _NS) == 0
    _, NB, CH = src3.shape
    RS = NP // _NS
    OP = ((CH + 15) // 16) * 16

    out_type = [jax.ShapeDtypeStruct((_NC, NP, D), jnp.float32)]
    scratch = [
        pltpu.VMEM_SHARED((NP, D), jnp.float32),
        pltpu.VMEM((NB, CH), jnp.int32),
        pltpu.VMEM((NB, CH), jnp.int32),
        pltpu.VMEM((128, D), jnp.float32),
        pltpu.VMEM((CH, D), jnp.float32),
        pltpu.VMEM((CH, D), jnp.float32),
        pltpu.SemaphoreType.DMA,
        pltpu.SemaphoreType.DMA,
    ]
    if want_deg:
        out_type.append(jax.ShapeDtypeStruct((_NC * NP,), jnp.float32))
        scratch.append(pltpu.VMEM_SHARED((NP,), jnp.float32))
        scratch.append(pltpu.VMEM((OP,), jnp.float32))
        scratch.append(pltpu.VMEM((RS,), jnp.float32))
        scratch.append(pltpu.SemaphoreType.DMA)

    mesh = plsc.VectorSubcoreMesh(core_axis_name="c", subcore_axis_name="s")

    def body(table_h, src_h, dst_h, out_h, *rest):
        if want_deg:
            (deg_h, acc_sh, src_v, dst_v, zbuf, b0, b1, semg, sems,
             deg_sh, ones_v, deg_v, semd) = rest
        else:
            acc_sh, src_v, dst_v, zbuf, b0, b1, semg, sems = rest
        c = lax.axis_index("c")
        s = lax.axis_index("s")
        wid = s * _NC + c
        r0 = s * RS

        zero16 = jnp.zeros((16,), jnp.float32)

        def zrow(i, carry):
            for k in range(D // 16):
                zbuf[i, pl.ds(k * 16, 16)] = zero16
            return carry

        lax.fori_loop(0, 128, zrow, 0)
        for q in range(RS // 128):
            pltpu.sync_copy(zbuf, acc_sh.at[pl.ds(r0 + q * 128, 128)])

        if want_deg:
            def zdeg(i, carry):
                deg_v[pl.ds(i * 16, 16)] = zero16
                return carry

            lax.fori_loop(0, RS // 16, zdeg, 0)
            pltpu.sync_copy(deg_v, deg_sh.at[pl.ds(r0, RS)])
            one16 = jnp.ones((16,), jnp.float32)
            for k in range(OP // 16):
                ones_v[pl.ds(k * 16, 16)] = one16

        pltpu.sync_copy(src_h.at[wid], src_v)
        pltpu.sync_copy(dst_h.at[wid], dst_v)

        plsc.subcore_barrier()

        bufs = [b0, b1]
        sd = [None] * NB
        gd = [None] * NB
        dd = [None] * NB
        gd[0] = pltpu.async_copy(table_h.at[src_v.at[0]], b0, semg)
        for j in range(NB):
            cur, nxt = bufs[j % 2], bufs[(j + 1) % 2]
            if j + 1 < NB:
                if j >= 1:
                    sd[j - 1].wait()
                gd[j + 1] = pltpu.async_copy(table_h.at[src_v.at[j + 1]],
                                             nxt, semg)
            gd[j].wait()
            sd[j] = pltpu.async_copy(cur, acc_sh.at[dst_v.at[j]], sems,
                                     add=True)
            if want_deg:
                dd[j] = pltpu.async_copy(ones_v.at[pl.ds(0, CH)],
                                         deg_sh.at[dst_v.at[j]], semd,
                                         add=True)
        sd[NB - 1].wait()
        if NB >= 2:
            sd[NB - 2].wait()
        if want_deg:
            for j in range(NB):
                dd[j].wait()

        plsc.subcore_barrier()

        pltpu.sync_copy(acc_sh.at[pl.ds(r0, RS)], out_h.at[c, pl.ds(r0, RS)])
        if want_deg:
            pltpu.sync_copy(deg_sh.at[pl.ds(r0, RS)], deg_v)
            pltpu.sync_copy(deg_v, deg_h.at[pl.ds(c * NP + r0, RS)])

    fn = pl.kernel(body, out_type=out_type, mesh=mesh, scratch_types=scratch,
                   compiler_params=pltpu.CompilerParams(
                       use_tc_tiling_on_sc=False))
    return fn(table, src3, dst3)


_BR = 10240


def _tc1(x, Wcat, bcat):
    NP, F = x.shape
    H2 = Wcat.shape[1]
    H = H2 // 2
    GN = NP // _BR

    def body(x_ref, w_ref, b_ref, y_ref, h1_ref):
        yh = jnp.dot(x_ref[...], w_ref[...],
                     preferred_element_type=jnp.float32) + b_ref[...]
        y_ref[...] = yh[:, :H]
        h1_ref[...] = yh[:, H:]

    return pl.pallas_call(
        body,
        grid=(GN,),
        in_specs=[pl.BlockSpec((_BR, F), lambda i: (i, 0)),
                  pl.BlockSpec((F, H2), lambda i: (0, 0)),
                  pl.BlockSpec((1, H2), lambda i: (0, 0))],
        out_specs=[pl.BlockSpec((_BR, H), lambda i: (i, 0)),
                   pl.BlockSpec((_BR, H), lambda i: (i, 0))],
        out_shape=[jax.ShapeDtypeStruct((NP, H), jnp.float32),
                   jax.ShapeDtypeStruct((NP, H), jnp.float32)],
    )(x, Wcat, bcat.reshape(1, H2))


def _tc2(y, h1, accA, degc, Tf2, tw, alpha):
    NP, H = y.shape
    T, M = tw.shape

    def body(y_ref, h1_ref, accA_ref, deg_ref, tf_ref, tw_ref, a_ref,
             yt_ref, yb_ref):
        a = a_ref[0, 0]
        deg = deg_ref[...]
        nbr = (accA_ref[0] + accA_ref[1]) / jnp.maximum(deg, 1.0)
        h = a * y_ref[...] + (1.0 - a) * nbr
        twv = tw_ref[...]
        twm = jnp.max(twv, axis=1, keepdims=True)
        we = jnp.exp(twv - twm)
        w = we / jnp.sum(we, axis=1, keepdims=True)
        tf = tf_ref[...]
        w_tile = jnp.tile(w, (1, T))
        col_t = lax.broadcasted_iota(jnp.int32, (T, T * M), 1) // M
        row_t = lax.broadcasted_iota(jnp.int32, (T, T * M), 0)
        B = jnp.where(col_t == row_t, w_tile, 0.0)
        Wt = jnp.dot(B, tf, preferred_element_type=jnp.float32)
        tn_row = jnp.sum(tf * tf, axis=1)[None, :]
        c1 = lax.dot_general(tn_row, B, (((1,), (1,)), ((), ())),
                             preferred_element_type=jnp.float32)
        hn = jnp.sum(h * h, axis=1, keepdims=True)
        cross = lax.dot_general(h, Wt, (((1,), (1,)), ((), ())),
                                preferred_element_type=jnp.float32)
        yt_ref[...] = hn + c1 - 2.0 * cross
        dinv = lax.rsqrt(deg + 1.0)
        yb_ref[...] = h1_ref[...] * dinv

    GN = NP // _BR
    return pl.pallas_call(
        body,
        grid=(GN,),
        in_specs=[pl.BlockSpec((_BR, H), lambda i: (i, 0)),
                  pl.BlockSpec((_BR, H), lambda i: (i, 0)),
                  pl.BlockSpec((2, _BR, H), lambda i: (0, i, 0)),
                  pl.BlockSpec((_BR, 1), lambda i: (i, 0)),
                  pl.BlockSpec(Tf2.shape, lambda i: (0, 0)),
                  pl.BlockSpec((T, M), lambda i: (0, 0)),
                  pl.BlockSpec((1, 1), lambda i: (0, 0))],
        out_specs=[pl.BlockSpec((_BR, T), lambda i: (i, 0)),
                   pl.BlockSpec((_BR, H), lambda i: (i, 0))],
        out_shape=[jax.ShapeDtypeStruct((NP, T), jnp.float32),
                   jax.ShapeDtypeStruct((NP, H), jnp.float32)],
    )(y, h1, accA, degc, Tf2, tw, alpha.reshape(1, 1))


def _tc3(accB, h1, degc, b1, yt, gamma, beta, W2, n_real):
    NP, H = h1.shape
    T = yt.shape[1]
    G = H + T
    C = W2.shape[1]
    CP = ((C + 15) // 16) * 16

    def body(accB_ref, h1_ref, deg_ref, b1_ref, yt_ref, g_ref, be_ref, w2_ref,
             xh_ref, h2_ref, h2b_ref):
        deg = deg_ref[...]
        dinv = lax.rsqrt(deg + 1.0)
        ssum = accB_ref[0] + accB_ref[1]
        z = jnp.maximum(dinv * ssum + h1_ref[...] * dinv * dinv + b1_ref[...],
                        0.0)
        xc = jnp.concatenate([z, yt_ref[...]], axis=1)
        xr = xc[:n_real]
        mu = jnp.mean(xr, axis=0, keepdims=True)
        d = xr - mu
        var = jnp.mean(d * d, axis=0, keepdims=True)
        xh = (xc - mu) * lax.rsqrt(var + 1e-5) * g_ref[...] + be_ref[...]
        xh_ref[...] = xh
        h2 = jnp.dot(xh, w2_ref[...], preferred_element_type=jnp.float32)
        h2_ref[...] = h2
        h2b = h2 * dinv
        h2b_ref[...] = jnp.concatenate(
            [h2b, jnp.zeros((h2b.shape[0], CP - C), jnp.float32)], axis=1)

    return pl.pallas_call(
        body,
        out_shape=[jax.ShapeDtypeStruct((NP, G), jnp.float32),
                   jax.ShapeDtypeStruct((NP, C), jnp.float32),
                   jax.ShapeDtypeStruct((NP, CP), jnp.float32)],
    )(accB, h1, degc, b1.reshape(1, H), yt, gamma.reshape(1, G),
      beta.reshape(1, G), W2)


def _tc4(accC, h2, degc, b2):
    NP, C = h2.shape
    CP = accC.shape[2]
    GN = NP // _BR

    def body(accC_ref, h2_ref, deg_ref, b2_ref, out_ref):
        deg = deg_ref[...]
        dinv = lax.rsqrt(deg + 1.0)
        acc = accC_ref[0, :, :C] + accC_ref[1, :, :C]
        out_ref[...] = dinv * acc + h2_ref[...] * dinv * dinv + b2_ref[...]

    return pl.pallas_call(
        body,
        grid=(GN,),
        in_specs=[pl.BlockSpec((2, _BR, CP), lambda i: (0, i, 0)),
                  pl.BlockSpec((_BR, C), lambda i: (i, 0)),
                  pl.BlockSpec((_BR, 1), lambda i: (i, 0)),
                  pl.BlockSpec((1, C), lambda i: (0, 0))],
        out_specs=pl.BlockSpec((_BR, C), lambda i: (i, 0)),
        out_shape=jax.ShapeDtypeStruct((NP, C), jnp.float32),
    )(accC, h2, degc, b2.reshape(1, C))


def kernel(x, edge_index, lin_W, lin_b, W1, b1, W2, b2, Tf, tw, alpha,
           gamma, beta):
    N, F = x.shape
    E = edge_index.shape[1]
    T, M, H = Tf.shape

    NP = ((N + 128 * _NS - 1) // (128 * _NS)) * (128 * _NS)
    EPT = E // _NW
    NCH = EPT // _CH
    src3 = edge_index[0].reshape(_NW, NCH, _CH)
    dst3 = edge_index[1].reshape(_NW, NCH, _CH)

    xp = jnp.pad(x, ((0, NP - N), (0, 0)))
    Wcat = jnp.concatenate([lin_W, W1], axis=1)
    bcat = jnp.concatenate([lin_b, jnp.zeros_like(b1)], axis=0)

    y, h1 = _tc1(xp, Wcat, bcat)
    accA, degp = _seg_sum(y, src3, dst3, want_deg=True)
    degc = (degp[:NP] + degp[NP:])[:, None]
    yt, yb = _tc2(y, h1, accA, degc, Tf.reshape(T * M, H), tw, alpha)
    (accB,) = _seg_sum(yb, src3, dst3, want_deg=False)
    xh, h2, h2b = _tc3(accB, h1, degc, b1, yt, gamma, beta, W2, N)
    (accCp,) = _seg_sum(h2b, src3, dst3, want_deg=False)
    out = _tc4(accCp, h2, degc, b2)
    return (out[:N], xh[:N])


# speedup vs baseline: 1.0637x; 1.0425x over previous
import functools

import jax
import jax.numpy as jnp
from jax import lax
from jax.experimental import pallas as pl
from jax.experimental.pallas import tpu as pltpu
from jax.experimental.pallas import tpu_sc as plsc

_NC = 2
_NS = 16
_NW = _NC * _NS
_CH = 400


def _seg_sum(table, src3, dst3, *, want_deg):
    NP, D = table.shape
    assert D % 16 == 0 and NP % (128 * _NS) == 0
    _, NB, CH = src3.shape
    RS = NP // _NS
    OP = ((CH + 15) // 16) * 16

    out_type = [jax.ShapeDtypeStruct((_NC, NP, D), jnp.float32)]
    scratch = [
        pltpu.VMEM_SHARED((NP, D), jnp.float32),
        pltpu.VMEM((NB, CH), jnp.int32),
        pltpu.VMEM((NB, CH), jnp.int32),
        pltpu.VMEM((128, D), jnp.float32),
        pltpu.VMEM((CH, D), jnp.float32),
        pltpu.VMEM((CH, D), jnp.float32),
        pltpu.SemaphoreType.DMA,
        pltpu.SemaphoreType.DMA,
    ]
    if want_deg:
        out_type.append(jax.ShapeDtypeStruct((_NC * NP,), jnp.float32))
        scratch.append(pltpu.VMEM_SHARED((NP,), jnp.float32))
        scratch.append(pltpu.VMEM((OP,), jnp.float32))
        scratch.append(pltpu.VMEM((RS,), jnp.float32))
        scratch.append(pltpu.SemaphoreType.DMA)

    mesh = plsc.VectorSubcoreMesh(core_axis_name="c", subcore_axis_name="s")

    def body(table_h, src_h, dst_h, out_h, *rest):
        if want_deg:
            (deg_h, acc_sh, src_v, dst_v, zbuf, b0, b1, semg, sems,
             deg_sh, ones_v, deg_v, semd) = rest
        else:
            acc_sh, src_v, dst_v, zbuf, b0, b1, semg, sems = rest
        c = lax.axis_index("c")
        s = lax.axis_index("s")
        wid = s * _NC + c
        r0 = s * RS

        zero16 = jnp.zeros((16,), jnp.float32)

        def zrow(i, carry):
            for k in range(D // 16):
                zbuf[i, pl.ds(k * 16, 16)] = zero16
            return carry

        lax.fori_loop(0, 128, zrow, 0)
        for q in range(RS // 128):
            pltpu.sync_copy(zbuf, acc_sh.at[pl.ds(r0 + q * 128, 128)])

        if want_deg:
            def zdeg(i, carry):
                deg_v[pl.ds(i * 16, 16)] = zero16
                return carry

            lax.fori_loop(0, RS // 16, zdeg, 0)
            pltpu.sync_copy(deg_v, deg_sh.at[pl.ds(r0, RS)])
            one16 = jnp.ones((16,), jnp.float32)
            for k in range(OP // 16):
                ones_v[pl.ds(k * 16, 16)] = one16

        pltpu.sync_copy(src_h.at[wid], src_v)
        pltpu.sync_copy(dst_h.at[wid], dst_v)

        plsc.subcore_barrier()

        bufs = [b0, b1]
        sd = [None] * NB
        gd = [None] * NB
        dd = [None] * NB
        gd[0] = pltpu.async_copy(table_h.at[src_v.at[0]], b0, semg)
        for j in range(NB):
            cur, nxt = bufs[j % 2], bufs[(j + 1) % 2]
            if j + 1 < NB:
                if j >= 1:
                    sd[j - 1].wait()
                gd[j + 1] = pltpu.async_copy(table_h.at[src_v.at[j + 1]],
                                             nxt, semg)
            gd[j].wait()
            sd[j] = pltpu.async_copy(cur, acc_sh.at[dst_v.at[j]], sems,
                                     add=True)
            if want_deg:
                dd[j] = pltpu.async_copy(ones_v.at[pl.ds(0, CH)],
                                         deg_sh.at[dst_v.at[j]], semd,
                                         add=True)
        sd[NB - 1].wait()
        if NB >= 2:
            sd[NB - 2].wait()
        if want_deg:
            for j in range(NB):
                dd[j].wait()

        plsc.subcore_barrier()

        pltpu.sync_copy(acc_sh.at[pl.ds(r0, RS)], out_h.at[c, pl.ds(r0, RS)])
        if want_deg:
            pltpu.sync_copy(deg_sh.at[pl.ds(r0, RS)], deg_v)
            pltpu.sync_copy(deg_v, deg_h.at[pl.ds(c * NP + r0, RS)])

    fn = pl.kernel(body, out_type=out_type, mesh=mesh, scratch_types=scratch,
                   compiler_params=pltpu.CompilerParams(
                       use_tc_tiling_on_sc=False))
    return fn(table, src3, dst3)


_BR = 10240


def _tc1(x, Wcat, bcat):
    NP, F = x.shape
    H2 = Wcat.shape[1]
    H = H2 // 2
    GN = NP // _BR

    def body(x_ref, w_ref, b_ref, y_ref, h1_ref):
        yh = jnp.dot(x_ref[...], w_ref[...],
                     preferred_element_type=jnp.float32) + b_ref[...]
        y_ref[...] = yh[:, :H]
        h1_ref[...] = yh[:, H:]

    return pl.pallas_call(
        body,
        grid=(GN,),
        in_specs=[pl.BlockSpec((_BR, F), lambda i: (i, 0)),
                  pl.BlockSpec((F, H2), lambda i: (0, 0)),
                  pl.BlockSpec((1, H2), lambda i: (0, 0))],
        out_specs=[pl.BlockSpec((_BR, H), lambda i: (i, 0)),
                   pl.BlockSpec((_BR, H), lambda i: (i, 0))],
        out_shape=[jax.ShapeDtypeStruct((NP, H), jnp.float32),
                   jax.ShapeDtypeStruct((NP, H), jnp.float32)],
    )(x, Wcat, bcat.reshape(1, H2))


def _tc2(y, h1, accA, degc, Tf2, tw, alpha):
    NP, H = y.shape
    T, M = tw.shape

    def body(y_ref, h1_ref, accA_ref, deg_ref, tf_ref, tw_ref, a_ref,
             yt_ref, yb_ref):
        a = a_ref[0, 0]
        deg = deg_ref[...]
        nbr = (accA_ref[0] + accA_ref[1]) / jnp.maximum(deg, 1.0)
        h = a * y_ref[...] + (1.0 - a) * nbr
        twv = tw_ref[...]
        twm = jnp.max(twv, axis=1, keepdims=True)
        we = jnp.exp(twv - twm)
        w = we / jnp.sum(we, axis=1, keepdims=True)
        tf = tf_ref[...]
        w_tile = jnp.tile(w, (1, T))
        col_t = lax.broadcasted_iota(jnp.int32, (T, T * M), 1) // M
        row_t = lax.broadcasted_iota(jnp.int32, (T, T * M), 0)
        B = jnp.where(col_t == row_t, w_tile, 0.0)
        Wt = jnp.dot(B, tf, preferred_element_type=jnp.float32)
        tn_row = jnp.sum(tf * tf, axis=1)[None, :]
        c1 = lax.dot_general(tn_row, B, (((1,), (1,)), ((), ())),
                             preferred_element_type=jnp.float32)
        hn = jnp.sum(h * h, axis=1, keepdims=True)
        cross = lax.dot_general(h, Wt, (((1,), (1,)), ((), ())),
                                preferred_element_type=jnp.float32)
        yt_ref[...] = hn + c1 - 2.0 * cross
        dinv = lax.rsqrt(deg + 1.0)
        yb_ref[...] = h1_ref[...] * dinv

    GN = NP // _BR
    return pl.pallas_call(
        body,
        grid=(GN,),
        in_specs=[pl.BlockSpec((_BR, H), lambda i: (i, 0)),
                  pl.BlockSpec((_BR, H), lambda i: (i, 0)),
                  pl.BlockSpec((2, _BR, H), lambda i: (0, i, 0)),
                  pl.BlockSpec((_BR, 1), lambda i: (i, 0)),
                  pl.BlockSpec(Tf2.shape, lambda i: (0, 0)),
                  pl.BlockSpec((T, M), lambda i: (0, 0)),
                  pl.BlockSpec((1, 1), lambda i: (0, 0))],
        out_specs=[pl.BlockSpec((_BR, T), lambda i: (i, 0)),
                   pl.BlockSpec((_BR, H), lambda i: (i, 0))],
        out_shape=[jax.ShapeDtypeStruct((NP, T), jnp.float32),
                   jax.ShapeDtypeStruct((NP, H), jnp.float32)],
    )(y, h1, accA, degc, Tf2, tw, alpha.reshape(1, 1))


def _tc3(accB, h1, degc, b1, yt, gamma, beta, W2, n_real):
    NP, H = h1.shape
    T = yt.shape[1]
    G = H + T
    C = W2.shape[1]
    CP = ((C + 15) // 16) * 16

    def body(accB_ref, h1_ref, deg_ref, b1_ref, yt_ref, g_ref, be_ref, w2_ref,
             xh_ref, h2_ref, h2b_ref):
        deg = deg_ref[...]
        dinv = lax.rsqrt(deg + 1.0)
        ssum = accB_ref[0] + accB_ref[1]
        z = jnp.maximum(dinv * ssum + h1_ref[...] * dinv * dinv + b1_ref[...],
                        0.0)
        xc = jnp.concatenate([z, yt_ref[...]], axis=1)
        xr = xc[:n_real]
        mu = jnp.mean(xr, axis=0, keepdims=True)
        d = xr - mu
        var = jnp.mean(d * d, axis=0, keepdims=True)
        xh = (xc - mu) * lax.rsqrt(var + 1e-5) * g_ref[...] + be_ref[...]
        xh_ref[...] = xh
        h2 = jnp.dot(xh, w2_ref[...], preferred_element_type=jnp.float32)
        h2_ref[...] = h2
        h2b = h2 * dinv
        h2b_ref[...] = jnp.concatenate(
            [h2b, jnp.zeros((h2b.shape[0], CP - C), jnp.float32)], axis=1)

    return pl.pallas_call(
        body,
        out_shape=[jax.ShapeDtypeStruct((NP, G), jnp.float32),
                   jax.ShapeDtypeStruct((NP, C), jnp.float32),
                   jax.ShapeDtypeStruct((NP, CP), jnp.float32)],
    )(accB, h1, degc, b1.reshape(1, H), yt, gamma.reshape(1, G),
      beta.reshape(1, G), W2)


def _tc4(accC, h2, degc, b2):
    NP, C = h2.shape
    CP = accC.shape[2]
    GN = NP // _BR

    def body(accC_ref, h2_ref, deg_ref, b2_ref, out_ref):
        deg = deg_ref[...]
        dinv = lax.rsqrt(deg + 1.0)
        acc = accC_ref[0, :, :C] + accC_ref[1, :, :C]
        out_ref[...] = dinv * acc + h2_ref[...] * dinv * dinv + b2_ref[...]

    return pl.pallas_call(
        body,
        grid=(GN,),
        in_specs=[pl.BlockSpec((2, _BR, CP), lambda i: (0, i, 0)),
                  pl.BlockSpec((_BR, C), lambda i: (i, 0)),
                  pl.BlockSpec((_BR, 1), lambda i: (i, 0)),
                  pl.BlockSpec((1, C), lambda i: (0, 0))],
        out_specs=pl.BlockSpec((_BR, C), lambda i: (i, 0)),
        out_shape=jax.ShapeDtypeStruct((NP, C), jnp.float32),
    )(accC, h2, degc, b2.reshape(1, C))


def kernel(x, edge_index, lin_W, lin_b, W1, b1, W2, b2, Tf, tw, alpha,
           gamma, beta):
    N, F = x.shape
    E = edge_index.shape[1]
    T, M, H = Tf.shape

    NP = ((N + 128 * _NS - 1) // (128 * _NS)) * (128 * _NS)
    EPT = E // _NW
    NCH = EPT // _CH
    src3 = edge_index[0].reshape(_NW, NCH, _CH)
    dst3 = edge_index[1].reshape(_NW, NCH, _CH)

    xp = jnp.pad(x, ((0, NP - N), (0, 0)))
    Wcat = jnp.concatenate([lin_W, W1], axis=1)
    bcat = jnp.concatenate([lin_b, jnp.zeros_like(b1)], axis=0)

    y, h1 = _tc1(xp, Wcat, bcat)
    accA, degp = _seg_sum(y, src3, dst3, want_deg=True)
    degc = (degp[:NP] + degp[NP:])[:, None]
    yt, yb = _tc2(y, h1, accA, degc, Tf.reshape(T * M, H), tw, alpha)
    (accB,) = _seg_sum(yb, src3, dst3, want_deg=False)
    xh, h2, h2b = _tc3(accB, h1, degc, b1, yt, gamma, beta, W2, N)
    (accCp,) = _seg_sum(h2b, src3, dst3, want_deg=False)
    out = _tc4(accCp, h2, degc, b2)
    return (out[:N], xh[:N])


# speedup vs baseline: 1.0646x; 1.0009x over previous
import jax
import jax.numpy as jnp
from jax import lax
from jax.experimental import pallas as pl
from jax.experimental.pallas import tpu as pltpu
from jax.experimental.pallas import tpu_sc as plsc

_NC = 2
_NS = 16
_NW = _NC * _NS
_CH = 400


def _seg_sum(table, src3, dst3, *, want_deg):
    NP, D = table.shape
    assert D % 16 == 0 and NP % (128 * _NS) == 0
    _, NB, CH = src3.shape
    RS = NP // _NS
    OP = ((CH + 15) // 16) * 16

    out_type = [jax.ShapeDtypeStruct((_NC, NP, D), jnp.float32)]
    scratch = [
        pltpu.VMEM_SHARED((NP, D), jnp.float32),
        pltpu.VMEM((NB, CH), jnp.int32),
        pltpu.VMEM((NB, CH), jnp.int32),
        pltpu.VMEM((128, D), jnp.float32),
        pltpu.VMEM((CH, D), jnp.float32),
        pltpu.VMEM((CH, D), jnp.float32),
        pltpu.SemaphoreType.DMA,
        pltpu.SemaphoreType.DMA,
    ]
    if want_deg:
        out_type.append(jax.ShapeDtypeStruct((_NC * NP,), jnp.float32))
        scratch.append(pltpu.VMEM_SHARED((NP,), jnp.float32))
        scratch.append(pltpu.VMEM((OP,), jnp.float32))
        scratch.append(pltpu.VMEM((RS,), jnp.float32))
        scratch.append(pltpu.SemaphoreType.DMA)

    mesh = plsc.VectorSubcoreMesh(core_axis_name="c", subcore_axis_name="s")

    def body(table_h, src_h, dst_h, out_h, *rest):
        if want_deg:
            (deg_h, acc_sh, src_v, dst_v, zbuf, b0, b1, semg, sems,
             deg_sh, ones_v, deg_v, semd) = rest
        else:
            acc_sh, src_v, dst_v, zbuf, b0, b1, semg, sems = rest
        c = lax.axis_index("c")
        s = lax.axis_index("s")
        wid = s * _NC + c
        r0 = s * RS

        zero16 = jnp.zeros((16,), jnp.float32)

        def zrow(i, carry):
            for k in range(D // 16):
                zbuf[i, pl.ds(k * 16, 16)] = zero16
            return carry

        lax.fori_loop(0, 128, zrow, 0)
        for q in range(RS // 128):
            pltpu.sync_copy(zbuf, acc_sh.at[pl.ds(r0 + q * 128, 128)])

        if want_deg:
            def zdeg(i, carry):
                deg_v[pl.ds(i * 16, 16)] = zero16
                return carry

            lax.fori_loop(0, RS // 16, zdeg, 0)
            pltpu.sync_copy(deg_v, deg_sh.at[pl.ds(r0, RS)])
            one16 = jnp.ones((16,), jnp.float32)
            for k in range(OP // 16):
                ones_v[pl.ds(k * 16, 16)] = one16

        pltpu.sync_copy(src_h.at[wid], src_v)
        pltpu.sync_copy(dst_h.at[wid], dst_v)

        plsc.subcore_barrier()

        bufs = [b0, b1]
        sd = [None] * NB
        gd = [None] * NB
        dd = [None] * NB
        gd[0] = pltpu.async_copy(table_h.at[src_v.at[0]], b0, semg)
        for j in range(NB):
            cur, nxt = bufs[j % 2], bufs[(j + 1) % 2]
            if j + 1 < NB:
                if j >= 1:
                    sd[j - 1].wait()
                gd[j + 1] = pltpu.async_copy(table_h.at[src_v.at[j + 1]],
                                             nxt, semg)
            gd[j].wait()
            sd[j] = pltpu.async_copy(cur, acc_sh.at[dst_v.at[j]], sems,
                                     add=True)
            if want_deg:
                dd[j] = pltpu.async_copy(ones_v.at[pl.ds(0, CH)],
                                         deg_sh.at[dst_v.at[j]], semd,
                                         add=True)
        sd[NB - 1].wait()
        if NB >= 2:
            sd[NB - 2].wait()
        if want_deg:
            for j in range(NB):
                dd[j].wait()

        plsc.subcore_barrier()

        pltpu.sync_copy(acc_sh.at[pl.ds(r0, RS)], out_h.at[c, pl.ds(r0, RS)])
        if want_deg:
            pltpu.sync_copy(deg_sh.at[pl.ds(r0, RS)], deg_v)
            pltpu.sync_copy(deg_v, deg_h.at[pl.ds(c * NP + r0, RS)])

    fn = pl.kernel(body, out_type=out_type, mesh=mesh, scratch_types=scratch,
                   compiler_params=pltpu.CompilerParams(
                       use_tc_tiling_on_sc=False))
    return fn(table, src3, dst3)


_BR = 10240


def _tc1(x, Wcat, bcat):
    NP, F = x.shape
    H2 = Wcat.shape[1]
    H = H2 // 2
    GN = NP // _BR

    def body(x_ref, w_ref, b_ref, y_ref, h1_ref):
        yh = jnp.dot(x_ref[...], w_ref[...],
                     preferred_element_type=jnp.float32) + b_ref[...]
        y_ref[...] = yh[:, :H]
        h1_ref[...] = yh[:, H:]

    return pl.pallas_call(
        body,
        grid=(GN,),
        in_specs=[pl.BlockSpec((_BR, F), lambda i: (i, 0)),
                  pl.BlockSpec((F, H2), lambda i: (0, 0)),
                  pl.BlockSpec((1, H2), lambda i: (0, 0))],
        out_specs=[pl.BlockSpec((_BR, H), lambda i: (i, 0)),
                   pl.BlockSpec((_BR, H), lambda i: (i, 0))],
        out_shape=[jax.ShapeDtypeStruct((NP, H), jnp.float32),
                   jax.ShapeDtypeStruct((NP, H), jnp.float32)],
    )(x, Wcat, bcat.reshape(1, H2))


def _tc2(y, h1, accA, degc, Tf2, tw, alpha):
    NP, H = y.shape
    T, M = tw.shape

    def body(y_ref, h1_ref, accA_ref, deg_ref, tf_ref, tw_ref, a_ref,
             yt_ref, yb_ref):
        a = a_ref[0, 0]
        deg = deg_ref[...]
        nbr = (accA_ref[0] + accA_ref[1]) / jnp.maximum(deg, 1.0)
        h = a * y_ref[...] + (1.0 - a) * nbr
        twv = tw_ref[...]
        twm = jnp.max(twv, axis=1, keepdims=True)
        we = jnp.exp(twv - twm)
        w = we / jnp.sum(we, axis=1, keepdims=True)
        tf = tf_ref[...]
        w_tile = jnp.tile(w, (1, T))
        col_t = lax.broadcasted_iota(jnp.int32, (T, T * M), 1) // M
        row_t = lax.broadcasted_iota(jnp.int32, (T, T * M), 0)
        B = jnp.where(col_t == row_t, w_tile, 0.0)
        Wt = jnp.dot(B, tf, preferred_element_type=jnp.float32)
        tn_row = jnp.sum(tf * tf, axis=1)[None, :]
        c1 = lax.dot_general(tn_row, B, (((1,), (1,)), ((), ())),
                             preferred_element_type=jnp.float32)
        hn = jnp.sum(h * h, axis=1, keepdims=True)
        cross = lax.dot_general(h, Wt, (((1,), (1,)), ((), ())),
                                preferred_element_type=jnp.float32)
        yt_ref[...] = hn + c1 - 2.0 * cross
        dinv = lax.rsqrt(deg + 1.0)
        yb_ref[...] = h1_ref[...] * dinv

    GN = NP // _BR
    return pl.pallas_call(
        body,
        grid=(GN,),
        in_specs=[pl.BlockSpec((_BR, H), lambda i: (i, 0)),
                  pl.BlockSpec((_BR, H), lambda i: (i, 0)),
                  pl.BlockSpec((2, _BR, H), lambda i: (0, i, 0)),
                  pl.BlockSpec((_BR, 1), lambda i: (i, 0)),
                  pl.BlockSpec(Tf2.shape, lambda i: (0, 0)),
                  pl.BlockSpec((T, M), lambda i: (0, 0)),
                  pl.BlockSpec((1, 1), lambda i: (0, 0))],
        out_specs=[pl.BlockSpec((_BR, T), lambda i: (i, 0)),
                   pl.BlockSpec((_BR, H), lambda i: (i, 0))],
        out_shape=[jax.ShapeDtypeStruct((NP, T), jnp.float32),
                   jax.ShapeDtypeStruct((NP, H), jnp.float32)],
    )(y, h1, accA, degc, Tf2, tw, alpha.reshape(1, 1))


def _tc3(accB, h1, degc, b1, yt, gamma, beta, W2, n_real):
    NP, H = h1.shape
    T = yt.shape[1]
    G = H + T
    C = W2.shape[1]
    CP = ((C + 15) // 16) * 16

    def body(accB_ref, h1_ref, deg_ref, b1_ref, yt_ref, g_ref, be_ref, w2_ref,
             xh_ref, h2_ref, h2b_ref):
        deg = deg_ref[...]
        dinv = lax.rsqrt(deg + 1.0)
        ssum = accB_ref[0] + accB_ref[1]
        z = jnp.maximum(dinv * ssum + h1_ref[...] * dinv * dinv + b1_ref[...],
                        0.0)
        xc = jnp.concatenate([z, yt_ref[...]], axis=1)
        xr = xc[:n_real]
        mu = jnp.mean(xr, axis=0, keepdims=True)
        d = xr - mu
        var = jnp.mean(d * d, axis=0, keepdims=True)
        xh = (xc - mu) * lax.rsqrt(var + 1e-5) * g_ref[...] + be_ref[...]
        xh_ref[...] = xh
        h2 = jnp.dot(xh, w2_ref[...], preferred_element_type=jnp.float32)
        h2_ref[...] = h2
        h2b = h2 * dinv
        h2b_ref[...] = jnp.concatenate(
            [h2b, jnp.zeros((h2b.shape[0], CP - C), jnp.float32)], axis=1)

    return pl.pallas_call(
        body,
        out_shape=[jax.ShapeDtypeStruct((NP, G), jnp.float32),
                   jax.ShapeDtypeStruct((NP, C), jnp.float32),
                   jax.ShapeDtypeStruct((NP, CP), jnp.float32)],
    )(accB, h1, degc, b1.reshape(1, H), yt, gamma.reshape(1, G),
      beta.reshape(1, G), W2)


def _tc4(accC, h2, degc, b2):
    NP, C = h2.shape
    CP = accC.shape[2]
    GN = NP // _BR

    def body(accC_ref, h2_ref, deg_ref, b2_ref, out_ref):
        deg = deg_ref[...]
        dinv = lax.rsqrt(deg + 1.0)
        acc = accC_ref[0, :, :C] + accC_ref[1, :, :C]
        out_ref[...] = dinv * acc + h2_ref[...] * dinv * dinv + b2_ref[...]

    return pl.pallas_call(
        body,
        grid=(GN,),
        in_specs=[pl.BlockSpec((2, _BR, CP), lambda i: (0, i, 0)),
                  pl.BlockSpec((_BR, C), lambda i: (i, 0)),
                  pl.BlockSpec((_BR, 1), lambda i: (i, 0)),
                  pl.BlockSpec((1, C), lambda i: (0, 0))],
        out_specs=pl.BlockSpec((_BR, C), lambda i: (i, 0)),
        out_shape=jax.ShapeDtypeStruct((NP, C), jnp.float32),
    )(accC, h2, degc, b2.reshape(1, C))


def kernel(x, edge_index, lin_W, lin_b, W1, b1, W2, b2, Tf, tw, alpha,
           gamma, beta):
    N, F = x.shape
    E = edge_index.shape[1]
    T, M, H = Tf.shape

    NP = ((N + 128 * _NS - 1) // (128 * _NS)) * (128 * _NS)
    EPT = E // _NW
    NCH = EPT // _CH
    src3 = edge_index[0].reshape(_NW, NCH, _CH)
    dst3 = edge_index[1].reshape(_NW, NCH, _CH)

    xp = jnp.pad(x, ((0, NP - N), (0, 0)))
    Wcat = jnp.concatenate([lin_W, W1], axis=1)
    bcat = jnp.concatenate([lin_b, jnp.zeros_like(b1)], axis=0)

    y, h1 = _tc1(xp, Wcat, bcat)
    accA, degp = _seg_sum(y, src3, dst3, want_deg=True)
    degc = (degp[:NP] + degp[NP:])[:, None]
    yt, yb = _tc2(y, h1, accA, degc, Tf.reshape(T * M, H), tw, alpha)
    (accB,) = _seg_sum(yb, src3, dst3, want_deg=False)
    xh, h2, h2b = _tc3(accB, h1, degc, b1, yt, gamma, beta, W2, N)
    (accCp,) = _seg_sum(h2b, src3, dst3, want_deg=False)
    out = _tc4(accCp, h2, degc, b2)
    return (out[:N], xh[:N])
